# Initial kernel scaffold; baseline (speedup 1.0000x reference)
import functools

import jax
import jax.numpy as jnp
from jax import lax
from jax.experimental import pallas as pl
from jax.experimental.pallas import tpu as pltpu
from jax.experimental.pallas import tpu_sc as plsc

VOCAB = 1000000
EMBED_DIM = 32
BATCH = 4096
HIST = 200

NC = 2
NS = 16
NW = NC * NS

LANE = 128
K = 8
TOTAL = BATCH * HIST
ROWS = TOTAL // LANE
RW = ROWS // NW
CHUNKS = RW // K


def _body(x_hbm, table_hbm, out_hbm, idx_v, rows_v, sem):
    wid = lax.axis_index("s") * NC + lax.axis_index("c")
    row0 = wid * RW

    def chunk(ci, carry):
        r = row0 + ci * K
        pltpu.sync_copy(x_hbm.at[pl.ds(r, K)], idx_v)
        descs = [
            pltpu.async_copy(
                table_hbm.at[idx_v.at[j]],
                rows_v.at[pl.ds(j * LANE, LANE)],
                sem,
            )
            for j in range(K)
        ]
        for d in descs:
            d.wait()
        pltpu.sync_copy(rows_v, out_hbm.at[pl.ds(r * LANE, K * LANE)])
        return carry

    lax.fori_loop(0, CHUNKS, chunk, 0)


@jax.jit
def _embed(x2d, table):
    mesh = plsc.VectorSubcoreMesh(core_axis_name="c", subcore_axis_name="s")
    fn = pl.kernel(
        _body,
        out_type=jax.ShapeDtypeStruct((TOTAL, EMBED_DIM), jnp.float32),
        mesh=mesh,
        scratch_types=[
            pltpu.VMEM((K, LANE), jnp.int32),
            pltpu.VMEM((K * LANE, EMBED_DIM), jnp.float32),
            pltpu.SemaphoreType.DMA,
        ],
    )
    return fn(x2d, table)


def kernel(x, table):
    x2d = x.reshape(ROWS, LANE).astype(jnp.int32)
    out = _embed(x2d, table)
    return out.reshape(BATCH, HIST, EMBED_DIM)


# speedup vs baseline: 1.4582x; 1.4582x over previous
import functools

import jax
import jax.numpy as jnp
from jax import lax
from jax.experimental import pallas as pl
from jax.experimental.pallas import tpu as pltpu
from jax.experimental.pallas import tpu_sc as plsc

VOCAB = 1000000
EMBED_DIM = 32
BATCH = 4096
HIST = 200

NC = 2
NS = 16
NW = NC * NS

LANE = 128
K = 8
TOTAL = BATCH * HIST
ROWS = TOTAL // LANE
RW = ROWS // NW
CHUNKS = RW // K


def _body(x_hbm, table_hbm, out_hbm, idx_v, rows_v, sem):
    wid = lax.axis_index("s") * NC + lax.axis_index("c")
    row0 = wid * RW

    def chunk(ci, carry):
        r = row0 + ci * K
        pltpu.sync_copy(x_hbm.at[pl.ds(r, K)], idx_v)
        descs = [
            pltpu.async_copy(
                table_hbm.at[idx_v.at[j]],
                rows_v.at[pl.ds(j * LANE, LANE)],
                sem,
            )
            for j in range(K)
        ]
        for d in descs:
            d.wait()
        pltpu.sync_copy(rows_v, out_hbm.at[pl.ds(r * LANE, K * LANE)])
        return carry

    lax.fori_loop(0, CHUNKS, chunk, 0)


@jax.jit
def _embed(x2d, table):
    mesh = plsc.VectorSubcoreMesh(core_axis_name="c", subcore_axis_name="s")
    fn = pl.kernel(
        _body,
        out_type=jax.ShapeDtypeStruct((TOTAL, EMBED_DIM), jnp.float32),
        mesh=mesh,
        scratch_types=[
            pltpu.VMEM((K, LANE), jnp.int32),
            pltpu.VMEM((K * LANE, EMBED_DIM), jnp.float32),
            pltpu.SemaphoreType.DMA,
        ],
        compiler_params=pltpu.CompilerParams(use_tc_tiling_on_sc=False),
    )
    return fn(x2d, table)


def kernel(x, table):
    x2d = x.reshape(ROWS, LANE).astype(jnp.int32)
    out = _embed(x2d, table)
    return out.reshape(BATCH, HIST, EMBED_DIM)


# speedup vs baseline: 1.5010x; 1.0293x over previous
import jax
import jax.numpy as jnp
from jax import lax
from jax.experimental import pallas as pl
from jax.experimental.pallas import tpu as pltpu
from jax.experimental.pallas import tpu_sc as plsc

VOCAB = 1000000
EMBED_DIM = 32
BATCH = 4096
HIST = 200

NC = 2
NS = 16
NW = NC * NS

LANE = 128
K = 10
TOTAL = BATCH * HIST
ROWS = TOTAL // LANE
RW = ROWS // NW
CHUNKS = RW // K
NB = 2


def _body(x_hbm, table_hbm, out_hbm, idx_v, rows0, rows1, gs0, gs1, ss0, ss1):
    rows_v = (rows0, rows1)
    gat_s = (gs0, gs1)
    st_s = (ss0, ss1)

    wid = lax.axis_index("s") * NC + lax.axis_index("c")
    row0 = wid * RW

    pltpu.sync_copy(x_hbm.at[pl.ds(row0, RW)], idx_v)

    def fire(g, b):
        for j in range(K):
            pltpu.async_copy(
                table_hbm.at[idx_v.at[g * K + j]],
                rows_v[b].at[pl.ds(j * LANE, LANE)],
                gat_s[b],
            )

    def drain_gat(g, b):
        pltpu.make_async_copy(
            out_hbm.at[pl.ds((row0 + g * K) * LANE, K * LANE)],
            rows_v[b],
            gat_s[b],
        ).wait()

    def start_store(g, b):
        pltpu.async_copy(
            rows_v[b],
            out_hbm.at[pl.ds((row0 + g * K) * LANE, K * LANE)],
            st_s[b],
        )

    def wait_store(g, b):
        pltpu.make_async_copy(
            rows_v[b],
            out_hbm.at[pl.ds((row0 + g * K) * LANE, K * LANE)],
            st_s[b],
        ).wait()

    fire(0, 0)
    fire(1, 1)
    drain_gat(0, 0)
    start_store(0, 0)

    def step(i, carry):
        g = 2 * i + 1
        wait_store(g - 1, 0)
        fire(g + 1, 0)
        drain_gat(g, 1)
        start_store(g, 1)
        wait_store(g, 1)
        fire(g + 2, 1)
        drain_gat(g + 1, 0)
        start_store(g + 1, 0)
        return carry

    lax.fori_loop(0, (CHUNKS - 2) // 2, step, 0)

    g = CHUNKS - 1
    drain_gat(g, 1)
    start_store(g, 1)
    wait_store(g - 1, 0)
    wait_store(g, 1)


@jax.jit
def _embed(x2d, table):
    mesh = plsc.VectorSubcoreMesh(core_axis_name="c", subcore_axis_name="s")
    fn = pl.kernel(
        _body,
        out_type=jax.ShapeDtypeStruct((TOTAL, EMBED_DIM), jnp.float32),
        mesh=mesh,
        scratch_types=[
            pltpu.VMEM((RW, LANE), jnp.int32),
            pltpu.VMEM((K * LANE, EMBED_DIM), jnp.float32),
            pltpu.VMEM((K * LANE, EMBED_DIM), jnp.float32),
            pltpu.SemaphoreType.DMA,
            pltpu.SemaphoreType.DMA,
            pltpu.SemaphoreType.DMA,
            pltpu.SemaphoreType.DMA,
        ],
        compiler_params=pltpu.CompilerParams(use_tc_tiling_on_sc=False),
    )
    return fn(x2d, table)


def kernel(x, table):
    x2d = x.reshape(ROWS, LANE).astype(jnp.int32)
    out = _embed(x2d, table)
    return out.reshape(BATCH, HIST, EMBED_DIM)
